# baseline (speedup 1.0000x reference)
import functools

import jax
import jax.numpy as jnp
from jax import lax
from jax.experimental import pallas as pl
from jax.experimental.pallas import tpu as pltpu
from jax.experimental.pallas import tpu_sc as plsc

NC, NS, L = 2, 16, 16
NW = NC * NS
F = 40
UNROLL = 4
ROW = 112
TPT = 80
EPS2 = 1e-24


def _rsqrt(s):
    i = lax.bitcast_convert_type(s, jnp.int32)
    y = lax.bitcast_convert_type(jnp.int32(0x5F3759DF) - (i >> 1), jnp.float32)
    hs = 0.5 * s
    y = y * (1.5 - hs * y * y)
    y = y * (1.5 - hs * y * y)
    return y


def _dot(a, b):
    return a[0] * b[0] + a[1] * b[1] + a[2] * b[2]


@functools.partial(jax.jit, static_argnames=("nf",))
def _sc_loss(tbl, faces_pad, nf):
    nchunk = nf // F
    mesh = plsc.VectorSubcoreMesh(core_axis_name="c", subcore_axis_name="s")

    @functools.partial(
        pl.kernel,
        mesh=mesh,
        out_type=jax.ShapeDtypeStruct((L, 3, nf), jnp.float32),
        scratch_types=[
            pltpu.VMEM((3 * F,), jnp.int32),
            pltpu.VMEM((3 * F, ROW), jnp.float32),
            pltpu.VMEM((3 * F, ROW), jnp.float32),
            pltpu.VMEM((L, 3, F), jnp.float32),
            pltpu.SemaphoreType.DMA,
            pltpu.SemaphoreType.DMA,
        ],
        compiler_params=pltpu.CompilerParams(
            use_tc_tiling_on_sc=False, needs_layout_passes=False
        ),
    )
    def k(tbl_hbm, face_hbm, out_hbm, idx_v, rows0, rows1, out_v, sg0, sg1):
        wid = lax.axis_index("s") * NC + lax.axis_index("c")
        lane = lax.iota(jnp.int32, 16)
        rows = (rows0, rows1)
        sg = (sg0, sg1)

        def idx_copy(t):
            c = wid + t * NW
            pltpu.sync_copy(face_hbm.at[pl.ds(c * (3 * F), 3 * F)], idx_v)

        def gat_desc(slot):
            return pltpu.make_async_copy(tbl_hbm.at[idx_v], rows[slot], sg[slot])

        def compute_chunk(t, rows_v):
            c = wid + t * NW

            @plsc.parallel_loop(0, F, 1, unroll=UNROLL)
            def one_face(j):
                r0 = 3 * j
                r1 = r0 + 1
                r2 = r0 + 2

                def ld(r, kk):
                    return rows_v[r, 16 * kk:16 * (kk + 1)]

                o0 = [ld(r0, kk) for kk in range(3)]
                o1 = [ld(r1, kk) for kk in range(3)]
                o2 = [ld(r2, kk) for kk in range(3)]
                g0 = [ld(r0, 3 + kk) for kk in range(3)]
                g1 = [ld(r1, 3 + kk) for kk in range(3)]
                g2 = [ld(r2, 3 + kk) for kk in range(3)]
                m = ld(r0, 6) * ld(r1, 6) * ld(r2, 6)

                e1 = [a - b for a, b in zip(o1, o0)]
                e2 = [a - b for a, b in zip(o2, o0)]
                e3 = [a - b for a, b in zip(e2, e1)]
                h1 = [a - b for a, b in zip(g1, g0)]
                h2 = [a - b for a, b in zip(g2, g0)]
                n = [h1[1] * h2[2] - h1[2] * h2[1],
                     h1[2] * h2[0] - h1[0] * h2[2],
                     h1[0] * h2[1] - h1[1] * h2[0]]

                snc = jnp.maximum(_dot(n, n), EPS2)
                d1 = _dot(e1, n)
                d2 = _dot(e2, n)
                d3 = d2 - d1
                c1 = jnp.abs(d1) * _rsqrt(jnp.maximum(_dot(e1, e1), EPS2) * snc) * m
                c2 = jnp.abs(d2) * _rsqrt(jnp.maximum(_dot(e2, e2), EPS2) * snc) * m
                c3 = jnp.abs(d3) * _rsqrt(jnp.maximum(_dot(e3, e3), EPS2) * snc) * m

                jv = jnp.full((16,), j, jnp.int32)
                plsc.store_scatter(out_v, [lane, jnp.full((16,), 0, jnp.int32), jv], c1)
                plsc.store_scatter(out_v, [lane, jnp.full((16,), 1, jnp.int32), jv], c2)
                plsc.store_scatter(out_v, [lane, jnp.full((16,), 2, jnp.int32), jv], c3)

            @pl.when(c < nchunk)
            def _():
                pltpu.sync_copy(out_v, out_hbm.at[:, :, pl.ds(c * F, F)])

        idx_copy(0)
        gat_desc(0).start()

        def body(t2, carry):
            t0 = 2 * t2
            t1 = t0 + 1
            gat_desc(0).wait()
            idx_copy(t1)
            gat_desc(1).start()
            compute_chunk(t0, rows0)
            gat_desc(1).wait()

            @pl.when(t1 + 1 < TPT)
            def _():
                idx_copy(t1 + 1)
                gat_desc(0).start()

            compute_chunk(t1, rows1)
            return carry

        lax.fori_loop(0, TPT // 2, body, 0)

    return k(tbl, faces_pad)


def kernel(coord_out, coord_gt, valid, face):
    B, V, D = coord_out.shape
    nf = face.shape[0]
    pad = TPT * NW * 3 * F - 3 * nf
    tbl = jnp.concatenate(
        [
            coord_out.transpose(1, 2, 0).reshape(V, D * B),
            coord_gt.transpose(1, 2, 0).reshape(V, D * B),
            valid[:, :, 0].T,
        ],
        axis=1,
    )
    faces_pad = jnp.concatenate(
        [face.reshape(-1), jnp.zeros((pad,), jnp.int32)])
    out = _sc_loss(tbl, faces_pad, nf)
    return out.reshape(B, 3 * nf, 1)


# speedup vs baseline: 1.3247x; 1.3247x over previous
import functools

import jax
import jax.numpy as jnp
from jax import lax
from jax.experimental import pallas as pl
from jax.experimental.pallas import tpu as pltpu
from jax.experimental.pallas import tpu_sc as plsc

NC, NS, L = 2, 16, 16
NW = NC * NS
F = 40
UNROLL = 4
ROW = 112
EPS2 = 1e-24


def _rsqrt(s):
    i = lax.bitcast_convert_type(s, jnp.int32)
    y = lax.bitcast_convert_type(jnp.int32(0x5F3759DF) - (i >> 1), jnp.float32)
    hs = 0.5 * s
    y = y * (1.5 - hs * y * y)
    y = y * (1.5 - hs * y * y)
    return y


def _dot(a, b):
    return a[0] * b[0] + a[1] * b[1] + a[2] * b[2]


@functools.partial(jax.jit, static_argnames=("nf",))
def _sc_loss(tbl, faces_flat, nf):
    nchunk = nf // F
    mesh = plsc.VectorSubcoreMesh(core_axis_name="c", subcore_axis_name="s")

    @functools.partial(
        pl.kernel,
        mesh=mesh,
        out_type=jax.ShapeDtypeStruct((L, 3, nf), jnp.float32),
        scratch_types=[
            pltpu.VMEM((3 * F,), jnp.int32),
            pltpu.VMEM((3 * F, ROW), jnp.float32),
            pltpu.VMEM((L, 3, F), jnp.float32),
            pltpu.SemaphoreType.DMA,
        ],
        compiler_params=pltpu.CompilerParams(
            use_tc_tiling_on_sc=False, needs_layout_passes=False
        ),
    )
    def k(tbl_hbm, face_hbm, out_hbm, idx_v, rows_v, out_v, sem):
        wid = lax.axis_index("s") * NC + lax.axis_index("c")
        my_chunks = (nchunk - wid + NW - 1) // NW
        lane = lax.iota(jnp.int32, 16)

        def chunk_body(t, carry):
            c = wid + t * NW
            pltpu.sync_copy(face_hbm.at[pl.ds(c * (3 * F), 3 * F)], idx_v)
            ga = pltpu.make_async_copy(
                tbl_hbm.at[idx_v.at[pl.ds(0, 64)]], rows_v.at[pl.ds(0, 64), :], sem)
            gb = pltpu.make_async_copy(
                tbl_hbm.at[idx_v.at[pl.ds(64, 56)]], rows_v.at[pl.ds(64, 56), :], sem)
            ga.start()
            gb.start()
            ga.wait()
            gb.wait()

            @plsc.parallel_loop(0, F, 1, unroll=UNROLL)
            def one_face(j):
                r0 = 3 * j
                r1 = r0 + 1
                r2 = r0 + 2

                def ld(r, kk):
                    return rows_v[r, 16 * kk:16 * (kk + 1)]

                o0 = [ld(r0, kk) for kk in range(3)]
                o1 = [ld(r1, kk) for kk in range(3)]
                o2 = [ld(r2, kk) for kk in range(3)]
                g0 = [ld(r0, 3 + kk) for kk in range(3)]
                g1 = [ld(r1, 3 + kk) for kk in range(3)]
                g2 = [ld(r2, 3 + kk) for kk in range(3)]
                m = ld(r0, 6) * ld(r1, 6) * ld(r2, 6)

                e1 = [a - b for a, b in zip(o1, o0)]
                e2 = [a - b for a, b in zip(o2, o0)]
                e3 = [a - b for a, b in zip(e2, e1)]
                h1 = [a - b for a, b in zip(g1, g0)]
                h2 = [a - b for a, b in zip(g2, g0)]
                n = [h1[1] * h2[2] - h1[2] * h2[1],
                     h1[2] * h2[0] - h1[0] * h2[2],
                     h1[0] * h2[1] - h1[1] * h2[0]]

                snc = jnp.maximum(_dot(n, n), EPS2)
                d1 = _dot(e1, n)
                d2 = _dot(e2, n)
                d3 = d2 - d1
                c1 = jnp.abs(d1) * _rsqrt(jnp.maximum(_dot(e1, e1), EPS2) * snc) * m
                c2 = jnp.abs(d2) * _rsqrt(jnp.maximum(_dot(e2, e2), EPS2) * snc) * m
                c3 = jnp.abs(d3) * _rsqrt(jnp.maximum(_dot(e3, e3), EPS2) * snc) * m

                jv = jnp.full((16,), j, jnp.int32)
                plsc.store_scatter(out_v, [lane, jnp.full((16,), 0, jnp.int32), jv], c1)
                plsc.store_scatter(out_v, [lane, jnp.full((16,), 1, jnp.int32), jv], c2)
                plsc.store_scatter(out_v, [lane, jnp.full((16,), 2, jnp.int32), jv], c3)

            pltpu.sync_copy(out_v, out_hbm.at[:, :, pl.ds(c * F, F)])
            return carry

        lax.fori_loop(0, my_chunks, chunk_body, 0)

    return k(tbl, faces_flat)


def kernel(coord_out, coord_gt, valid, face):
    B, V, D = coord_out.shape
    nf = face.shape[0]
    tbl = jnp.concatenate(
        [
            coord_out.transpose(1, 2, 0).reshape(V, D * B),
            coord_gt.transpose(1, 2, 0).reshape(V, D * B),
            valid[:, :, 0].T,
        ],
        axis=1,
    )
    out = _sc_loss(tbl, face.reshape(-1), nf)
    return out.reshape(B, 3 * nf, 1)
